# baseline (speedup 1.0000x reference)
import functools

import jax
import jax.numpy as jnp
from jax import lax
from jax.experimental import pallas as pl
from jax.experimental.pallas import tpu as pltpu
from jax.experimental.pallas import tpu_sc as plsc

N = 10000
E = 320000
NC = 2
NS = 16
NW = NC * NS
CH = 128
NCHUNK = 80
EPW = NCHUNK * CH
E_PAD = NW * EPW
WT = 10
RPW = N // WT
DUMMY = 256
H0 = 128
NP_DEG = 10496
DPW = NP_DEG // NS

BN = 1000
NB = N // BN


def _sc_agg(table, pk3, with_deg):
    W = table.shape[1]
    K = 2
    mesh = plsc.VectorSubcoreMesh(core_axis_name="c", subcore_axis_name="s",
                                  num_cores=NC, num_subcores=NS)
    out_type = [jax.ShapeDtypeStruct((NC, N, W), jnp.float32)]
    if with_deg:
        out_type.append(jax.ShapeDtypeStruct((NC * NP_DEG,), jnp.float32))
    scratch = [
        pltpu.VMEM((NCHUNK, CH), jnp.int32),
        pltpu.VMEM((K, CH), jnp.int32),
        pltpu.VMEM((K, CH), jnp.int32),
        pltpu.VMEM((K, CH, W), jnp.float32),
        pltpu.VMEM((CH,), jnp.float32),
        pltpu.VMEM((DPW,), jnp.float32),
        pltpu.VMEM_SHARED((N + DUMMY, W), jnp.float32),
        pltpu.VMEM_SHARED((NP_DEG,), jnp.float32),
    ] + [pltpu.SemaphoreType.DMA] * (3 * K)

    def body(table_h, pk_h, acc_h, *rest):
        if with_deg:
            deg_h = rest[0]
            rest = rest[1:]
        pbuf, sidx, didx, rows, ones_v, degb, acc_sh, deg_sh = rest[:8]
        sem_g = rest[8:8 + K]
        sem_s = rest[8 + K:8 + 2 * K]
        sem_d = rest[8 + 2 * K:8 + 3 * K]
        c = lax.axis_index("c")
        s = lax.axis_index("s")
        wid = c * NS + s
        zero16 = jnp.zeros((16,), jnp.float32)
        per_row = W // 16

        pltpu.sync_copy(pk_h.at[wid], pbuf)

        def zr(i, _):
            rows[0, i // per_row, pl.ds((i % per_row) * 16, 16)] = zero16
            return 0
        lax.fori_loop(0, CH * per_row, zr, 0)

        def zd(i, _):
            degb[pl.ds(i * 16, 16)] = zero16
            return 0
        lax.fori_loop(0, DPW // 16, zd, 0)

        def of(i, _):
            ones_v[pl.ds(i * 16, 16)] = jnp.ones((16,), jnp.float32)
            return 0
        lax.fori_loop(0, CH // 16, of, 0)

        @pl.when(s < WT)
        def _clear():
            for m in range(8):
                sz = 128 if m < 7 else 104
                pltpu.sync_copy(rows.at[0, pl.ds(0, sz)],
                                acc_sh.at[pl.ds(s * RPW + m * 128, sz)])
        if with_deg:
            pltpu.sync_copy(degb, deg_sh.at[pl.ds(s * DPW, DPW)])

        def unpack(k, j):
            def u(i, _):
                p = pbuf[j, pl.ds(i * 16, 16)]
                sidx[k, pl.ds(i * 16, 16)] = p & 16383
                didx[k, pl.ds(i * 16, 16)] = lax.shift_right_logical(p, 14)
                return 0
            lax.fori_loop(0, CH // 16, u, 0)

        def gchunk(k, j):
            pltpu.async_copy(table_h.at[sidx.at[k]], rows.at[k], sem_g[k])

        def wg(k):
            pltpu.make_async_copy(table_h.at[sidx.at[k]], rows.at[k],
                                  sem_g[k]).wait()

        def schunk(k):
            pltpu.async_copy(rows.at[k], acc_sh.at[didx.at[k]], sem_s[k],
                             add=True)
            if with_deg:
                pltpu.async_copy(ones_v, deg_sh.at[didx.at[k]], sem_d[k],
                                 add=True)

        def ws(k):
            pltpu.make_async_copy(rows.at[k], acc_sh.at[didx.at[k]],
                                  sem_s[k]).wait()
            if with_deg:
                pltpu.make_async_copy(ones_v, deg_sh.at[didx.at[k]],
                                      sem_d[k]).wait()

        unpack(0, 0)
        gchunk(0, 0)
        plsc.subcore_barrier()
        wg(0)
        schunk(0)
        unpack(1, 1)
        gchunk(1, 1)

        def group(g, _):
            for k in range(2):
                j = 1 + g * 2 + k
                a = 1 - k
                wg(a)
                schunk(a)
                ws(1 - a)
                unpack(1 - a, j + 1)
                gchunk(1 - a, j + 1)
            return 0
        lax.fori_loop(0, (NCHUNK - 2) // 2, group, 0)
        wg(1)
        schunk(1)
        ws(0)
        ws(1)
        plsc.subcore_barrier()

        @pl.when(s < WT)
        def _writeback():
            pltpu.sync_copy(acc_sh.at[pl.ds(s * RPW, RPW)],
                            acc_h.at[c, pl.ds(s * RPW, RPW)])
        if with_deg:
            pltpu.sync_copy(deg_sh.at[pl.ds(s * DPW, DPW)], degb)
            pltpu.sync_copy(degb, deg_h.at[pl.ds(c * NP_DEG + s * DPW, DPW)])

    res = pl.kernel(
        body,
        out_type=out_type,
        mesh=mesh,
        scratch_types=scratch,
        compiler_params=pltpu.CompilerParams(use_tc_tiling_on_sc=False),
        name=f"sc_agg_w{W}" + ("_deg" if with_deg else ""),
    )(table, pk3)
    return res if with_deg else res[0]


def _mm_t(a, w):
    return lax.dot_general(a, w, (((1,), (1,)), ((), ())),
                           preferred_element_type=jnp.float32)


def _tc_self(h, ws, b):
    H = ws.shape[0]

    def body(x_ref, ws_ref, b_ref, o_ref):
        o_ref[...] = _mm_t(x_ref[...], ws_ref[...]) + b_ref[...]

    return pl.pallas_call(
        body,
        grid=(NB,),
        in_specs=[
            pl.BlockSpec((BN, h.shape[1]), lambda i: (i, 0)),
            pl.BlockSpec(ws.shape, lambda i: (0, 0)),
            pl.BlockSpec((1, H), lambda i: (0, 0)),
        ],
        out_specs=pl.BlockSpec((BN, H), lambda i: (i, 0)),
        out_shape=jax.ShapeDtypeStruct((N, H), jnp.float32),
    )(h, ws, b.reshape(1, H))


def _full(w):
    return pl.BlockSpec(w.shape, lambda i: (0, 0))


def _rows(width):
    return pl.BlockSpec((BN, width), lambda i: (i, 0))


def _mean(d_ref, a_ref):
    d = d_ref[0, 0, :] + d_ref[0, 1, :]
    recip = 1.0 / jnp.maximum(d, 1.0)
    return (a_ref[0] + a_ref[1]) * recip[:, None]


_DEG_SPEC = pl.BlockSpec((1, NC, BN), lambda i: (i, 0, 0))


def _tc_c0(deg3, s0, acc0, wn0, ws1, b1):
    def body(d_ref, s_ref, a_ref, wn_ref, ws_ref, b_ref, h_ref, s1_ref):
        h = jnp.maximum(s_ref[...] + _mm_t(_mean(d_ref, a_ref), wn_ref[...]),
                        0.0)
        h_ref[...] = h
        s1_ref[...] = _mm_t(h, ws_ref[...]) + b_ref[...]

    return pl.pallas_call(
        body,
        grid=(NB,),
        in_specs=[_DEG_SPEC, _rows(H0), pl.BlockSpec((NC, BN, H0),
                  lambda i: (0, i, 0)), _full(wn0), _full(ws1),
                  pl.BlockSpec((1, H0), lambda i: (0, 0))],
        out_specs=[_rows(H0), _rows(H0)],
        out_shape=[jax.ShapeDtypeStruct((N, H0), jnp.float32)] * 2,
    )(deg3, s0, acc0, wn0, ws1, b1.reshape(1, H0))


def _tc_c1(deg3, s1, acc1, wn1, wn2, ws2, b2):
    C = wn2.shape[0]

    def body(d_ref, s_ref, a_ref, wn_ref, wt_ref, ws_ref, b_ref,
             t_ref, s2_ref):
        h = jnp.maximum(s_ref[...] + _mm_t(_mean(d_ref, a_ref), wn_ref[...]),
                        0.0)
        t_ref[...] = _mm_t(h, wt_ref[...])
        s2_ref[...] = _mm_t(h, ws_ref[...]) + b_ref[...]

    return pl.pallas_call(
        body,
        grid=(NB,),
        in_specs=[_DEG_SPEC, _rows(H0), pl.BlockSpec((NC, BN, H0),
                  lambda i: (0, i, 0)), _full(wn1), _full(wn2), _full(ws2),
                  pl.BlockSpec((1, C), lambda i: (0, 0))],
        out_specs=[_rows(C), _rows(C)],
        out_shape=[jax.ShapeDtypeStruct((N, C), jnp.float32)] * 2,
    )(deg3, s1, acc1, wn1, wn2, ws2, b2.reshape(1, C))


def _tc_c2(deg3, s2, acc2):
    C = s2.shape[1]

    def body(d_ref, s_ref, a_ref, o_ref):
        o_ref[...] = s_ref[...] + _mean(d_ref, a_ref)

    return pl.pallas_call(
        body,
        grid=(NB,),
        in_specs=[_DEG_SPEC, _rows(C),
                  pl.BlockSpec((NC, BN, C), lambda i: (0, i, 0))],
        out_specs=_rows(C),
        out_shape=jax.ShapeDtypeStruct((N, C), jnp.float32),
    )(deg3, s2, acc2)


def kernel(x, edge_index, W_neigh0, W_self0, b0, W_neigh1, W_self1, b1,
           W_neigh2, W_self2, b2):
    src = edge_index[0].astype(jnp.int32)
    dst = edge_index[1].astype(jnp.int32)
    pad = E_PAD - E
    packed = src | (dst << 14)
    spread = jnp.arange(pad, dtype=jnp.int32) % DUMMY
    pk3 = jnp.concatenate(
        [packed, spread | ((N + spread) << 14)]).reshape(NW, NCHUNK, CH)

    s0 = _tc_self(x, W_self0, b0)
    acc0, degf = _sc_agg(x, pk3, with_deg=True)
    deg3 = degf.reshape(NC, NP_DEG)[:, :N].reshape(NC, NB, BN).transpose(1, 0, 2)

    h0, s1 = _tc_c0(deg3, s0, acc0, W_neigh0, W_self1, b1)
    acc1 = _sc_agg(h0, pk3, with_deg=False)
    t, s2 = _tc_c1(deg3, s1, acc1, W_neigh1, W_neigh2, W_self2, b2)
    acc2 = _sc_agg(t, pk3, with_deg=False)
    return _tc_c2(deg3, s2, acc2)


# speedup vs baseline: 1.2432x; 1.2432x over previous
import functools

import jax
import jax.numpy as jnp
from jax import lax
from jax.experimental import pallas as pl
from jax.experimental.pallas import tpu as pltpu
from jax.experimental.pallas import tpu_sc as plsc

N = 10000
E = 320000
NC = 2
NS = 16
NW = NC * NS
CH = 128
NCHUNK = 80
EPW = NCHUNK * CH
E_PAD = NW * EPW
WT = 10
RPW = N // WT
DUMMY = 256
H0 = 128
NP_DEG = 10496
DPW = NP_DEG // NS

BN = 1000
NB = N // BN


def _sc_agg(table, pk3, with_deg):
    W = table.shape[1]
    K = 2 if W == 128 else 4
    NGRP = NCHUNK // K
    mesh = plsc.VectorSubcoreMesh(core_axis_name="c", subcore_axis_name="s",
                                  num_cores=NC, num_subcores=NS)
    out_type = [jax.ShapeDtypeStruct((NC, N, W), jnp.float32)]
    if with_deg:
        out_type.append(jax.ShapeDtypeStruct((NC * NP_DEG,), jnp.float32))
    scratch = [
        pltpu.VMEM((NCHUNK, CH), jnp.int32),
        pltpu.VMEM((K, CH), jnp.int32),
        pltpu.VMEM((K, CH), jnp.int32),
        pltpu.VMEM((K, CH, W), jnp.float32),
        pltpu.VMEM((CH,), jnp.float32),
        pltpu.VMEM((DPW,), jnp.float32),
        pltpu.VMEM_SHARED((N + DUMMY, W), jnp.float32),
        pltpu.VMEM_SHARED((NP_DEG,), jnp.float32),
    ] + [pltpu.SemaphoreType.DMA] * (3 * K)

    def body(table_h, pk_h, acc_h, *rest):
        if with_deg:
            deg_h = rest[0]
            rest = rest[1:]
        pbuf, sidx, didx, rows, ones_v, degb, acc_sh, deg_sh = rest[:8]
        sem_g = rest[8:8 + K]
        sem_s = rest[8 + K:8 + 2 * K]
        sem_d = rest[8 + 2 * K:8 + 3 * K]
        c = lax.axis_index("c")
        s = lax.axis_index("s")
        wid = c * NS + s
        zero16 = jnp.zeros((16,), jnp.float32)
        per_row = W // 16

        pltpu.sync_copy(pk_h.at[wid], pbuf)

        def zr(i, _):
            rows[0, i // per_row, pl.ds((i % per_row) * 16, 16)] = zero16
            return 0
        lax.fori_loop(0, CH * per_row, zr, 0)

        def zd(i, _):
            degb[pl.ds(i * 16, 16)] = zero16
            return 0
        lax.fori_loop(0, DPW // 16, zd, 0)

        def of(i, _):
            ones_v[pl.ds(i * 16, 16)] = jnp.ones((16,), jnp.float32)
            return 0
        lax.fori_loop(0, CH // 16, of, 0)

        @pl.when(s < WT)
        def _clear():
            for m in range(8):
                sz = 128 if m < 7 else 104
                pltpu.sync_copy(rows.at[0, pl.ds(0, sz)],
                                acc_sh.at[pl.ds(s * RPW + m * 128, sz)])
        if with_deg:
            pltpu.sync_copy(degb, deg_sh.at[pl.ds(s * DPW, DPW)])

        def unpack(k, j):
            def u(i, _):
                p = pbuf[j, pl.ds(i * 16, 16)]
                sidx[k, pl.ds(i * 16, 16)] = p & 16383
                didx[k, pl.ds(i * 16, 16)] = lax.shift_right_logical(p, 14)
                return 0
            lax.fori_loop(0, CH // 16, u, 0)

        for k in range(K):
            unpack(k, k)
            pltpu.async_copy(table_h.at[sidx.at[k]], rows.at[k], sem_g[k])
        plsc.subcore_barrier()

        def group(g, _):
            for k in range(K):
                pltpu.make_async_copy(table_h.at[sidx.at[k]], rows.at[k],
                                      sem_g[k]).wait()
                pltpu.async_copy(rows.at[k], acc_sh.at[didx.at[k]], sem_s[k],
                                 add=True)
                if with_deg:
                    pltpu.async_copy(ones_v, deg_sh.at[didx.at[k]], sem_d[k],
                                     add=True)

                @pl.when(g < NGRP - 1)
                def _refill():
                    pltpu.make_async_copy(rows.at[k], acc_sh.at[didx.at[k]],
                                          sem_s[k]).wait()
                    if with_deg:
                        pltpu.make_async_copy(ones_v, deg_sh.at[didx.at[k]],
                                              sem_d[k]).wait()
                    unpack(k, g * K + k + K)
                    pltpu.async_copy(table_h.at[sidx.at[k]], rows.at[k],
                                     sem_g[k])
            return 0
        lax.fori_loop(0, NGRP, group, 0)
        for k in range(K):
            pltpu.make_async_copy(rows.at[k], acc_sh.at[didx.at[k]],
                                  sem_s[k]).wait()
            if with_deg:
                pltpu.make_async_copy(ones_v, deg_sh.at[didx.at[k]],
                                      sem_d[k]).wait()
        plsc.subcore_barrier()

        @pl.when(s < WT)
        def _writeback():
            pltpu.sync_copy(acc_sh.at[pl.ds(s * RPW, RPW)],
                            acc_h.at[c, pl.ds(s * RPW, RPW)])
        if with_deg:
            pltpu.sync_copy(deg_sh.at[pl.ds(s * DPW, DPW)], degb)
            pltpu.sync_copy(degb, deg_h.at[pl.ds(c * NP_DEG + s * DPW, DPW)])

    res = pl.kernel(
        body,
        out_type=out_type,
        mesh=mesh,
        scratch_types=scratch,
        compiler_params=pltpu.CompilerParams(use_tc_tiling_on_sc=False),
        name=f"sc_agg_w{W}" + ("_deg" if with_deg else ""),
    )(table, pk3)
    return res if with_deg else res[0]


def _mm_t(a, w):
    return lax.dot_general(a, w, (((1,), (1,)), ((), ())),
                           preferred_element_type=jnp.float32)


def _tc_self(h, ws, b):
    H = ws.shape[0]

    def body(x_ref, ws_ref, b_ref, o_ref):
        o_ref[...] = _mm_t(x_ref[...], ws_ref[...]) + b_ref[...]

    return pl.pallas_call(
        body,
        grid=(NB,),
        in_specs=[
            pl.BlockSpec((BN, h.shape[1]), lambda i: (i, 0)),
            pl.BlockSpec(ws.shape, lambda i: (0, 0)),
            pl.BlockSpec((1, H), lambda i: (0, 0)),
        ],
        out_specs=pl.BlockSpec((BN, H), lambda i: (i, 0)),
        out_shape=jax.ShapeDtypeStruct((N, H), jnp.float32),
    )(h, ws, b.reshape(1, H))


def _full(w):
    return pl.BlockSpec(w.shape, lambda i: (0, 0))


def _rows(width):
    return pl.BlockSpec((BN, width), lambda i: (i, 0))


def _mean(d_ref, a_ref):
    d = d_ref[0, 0, :] + d_ref[0, 1, :]
    recip = 1.0 / jnp.maximum(d, 1.0)
    return (a_ref[0] + a_ref[1]) * recip[:, None]


_DEG_SPEC = pl.BlockSpec((1, NC, BN), lambda i: (i, 0, 0))


def _tc_c0(deg3, s0, acc0, wn0, ws1, b1):
    def body(d_ref, s_ref, a_ref, wn_ref, ws_ref, b_ref, h_ref, s1_ref):
        h = jnp.maximum(s_ref[...] + _mm_t(_mean(d_ref, a_ref), wn_ref[...]),
                        0.0)
        h_ref[...] = h
        s1_ref[...] = _mm_t(h, ws_ref[...]) + b_ref[...]

    return pl.pallas_call(
        body,
        grid=(NB,),
        in_specs=[_DEG_SPEC, _rows(H0), pl.BlockSpec((NC, BN, H0),
                  lambda i: (0, i, 0)), _full(wn0), _full(ws1),
                  pl.BlockSpec((1, H0), lambda i: (0, 0))],
        out_specs=[_rows(H0), _rows(H0)],
        out_shape=[jax.ShapeDtypeStruct((N, H0), jnp.float32)] * 2,
    )(deg3, s0, acc0, wn0, ws1, b1.reshape(1, H0))


def _tc_c1(deg3, s1, acc1, wn1, wn2, ws2, b2):
    C = wn2.shape[0]

    def body(d_ref, s_ref, a_ref, wn_ref, wt_ref, ws_ref, b_ref,
             t_ref, s2_ref):
        h = jnp.maximum(s_ref[...] + _mm_t(_mean(d_ref, a_ref), wn_ref[...]),
                        0.0)
        t_ref[...] = _mm_t(h, wt_ref[...])
        s2_ref[...] = _mm_t(h, ws_ref[...]) + b_ref[...]

    return pl.pallas_call(
        body,
        grid=(NB,),
        in_specs=[_DEG_SPEC, _rows(H0), pl.BlockSpec((NC, BN, H0),
                  lambda i: (0, i, 0)), _full(wn1), _full(wn2), _full(ws2),
                  pl.BlockSpec((1, C), lambda i: (0, 0))],
        out_specs=[_rows(C), _rows(C)],
        out_shape=[jax.ShapeDtypeStruct((N, C), jnp.float32)] * 2,
    )(deg3, s1, acc1, wn1, wn2, ws2, b2.reshape(1, C))


def _tc_c2(deg3, s2, acc2):
    C = s2.shape[1]

    def body(d_ref, s_ref, a_ref, o_ref):
        o_ref[...] = s_ref[...] + _mean(d_ref, a_ref)

    return pl.pallas_call(
        body,
        grid=(NB,),
        in_specs=[_DEG_SPEC, _rows(C),
                  pl.BlockSpec((NC, BN, C), lambda i: (0, i, 0))],
        out_specs=_rows(C),
        out_shape=jax.ShapeDtypeStruct((N, C), jnp.float32),
    )(deg3, s2, acc2)


def kernel(x, edge_index, W_neigh0, W_self0, b0, W_neigh1, W_self1, b1,
           W_neigh2, W_self2, b2):
    src = edge_index[0].astype(jnp.int32)
    dst = edge_index[1].astype(jnp.int32)
    pad = E_PAD - E
    packed = src | (dst << 14)
    spread = jnp.arange(pad, dtype=jnp.int32) % DUMMY
    pk3 = jnp.concatenate(
        [packed, spread | ((N + spread) << 14)]).reshape(NW, NCHUNK, CH)

    s0 = _tc_self(x, W_self0, b0)
    acc0, degf = _sc_agg(x, pk3, with_deg=True)
    deg3 = degf.reshape(NC, NP_DEG)[:, :N].reshape(NC, NB, BN).transpose(1, 0, 2)

    h0, s1 = _tc_c0(deg3, s0, acc0, W_neigh0, W_self1, b1)
    acc1 = _sc_agg(h0, pk3, with_deg=False)
    t, s2 = _tc_c1(deg3, s1, acc1, W_neigh1, W_neigh2, W_self2, b2)
    acc2 = _sc_agg(t, pk3, with_deg=False)
    return _tc_c2(deg3, s2, acc2)


# speedup vs baseline: 1.2715x; 1.0228x over previous
import functools

import jax
import jax.numpy as jnp
from jax import lax
from jax.experimental import pallas as pl
from jax.experimental.pallas import tpu as pltpu
from jax.experimental.pallas import tpu_sc as plsc

N = 10000
E = 320000
NC = 2
NS = 16
NW = NC * NS
CH = 128
NCHUNK = 80
EPW = NCHUNK * CH
E_PAD = NW * EPW
WT = 10
RPW = N // WT
DUMMY = 256
H0 = 128
NP_DEG = 10496
DPW = NP_DEG // NS

BN = 1000
NB = N // BN


def _sc_agg(table, pk3, with_deg):
    W = table.shape[1]
    K = 2 if W == 128 else 4
    NGRP = NCHUNK // K
    mesh = plsc.VectorSubcoreMesh(core_axis_name="c", subcore_axis_name="s",
                                  num_cores=NC, num_subcores=NS)
    out_type = [jax.ShapeDtypeStruct((NC, N, W), jnp.float32)]
    if with_deg:
        out_type.append(jax.ShapeDtypeStruct((NC * NP_DEG,), jnp.float32))
    scratch = [
        pltpu.VMEM((NCHUNK, CH), jnp.int32),
        pltpu.VMEM((K, CH), jnp.int32),
        pltpu.VMEM((K, CH), jnp.int32),
        pltpu.VMEM((K, CH, W), jnp.float32),
        pltpu.VMEM((CH,), jnp.float32),
        pltpu.VMEM((DPW,), jnp.float32),
        pltpu.VMEM_SHARED((N + DUMMY, W), jnp.float32),
        pltpu.VMEM_SHARED((NP_DEG,), jnp.float32),
    ] + [pltpu.SemaphoreType.DMA] * (3 * K)

    def body(table_h, pk_h, acc_h, *rest):
        if with_deg:
            deg_h = rest[0]
            rest = rest[1:]
        pbuf, sidx, didx, rows, ones_v, degb, acc_sh, deg_sh = rest[:8]
        sem_g = rest[8:8 + K]
        sem_s = rest[8 + K:8 + 2 * K]
        sem_d = rest[8 + 2 * K:8 + 3 * K]
        c = lax.axis_index("c")
        s = lax.axis_index("s")
        wid = c * NS + s
        zero16 = jnp.zeros((16,), jnp.float32)
        per_row = W // 16

        pltpu.sync_copy(pk_h.at[wid], pbuf)

        def zr(i, _):
            rows[0, i // per_row, pl.ds((i % per_row) * 16, 16)] = zero16
            return 0
        lax.fori_loop(0, CH * per_row, zr, 0)

        def zd(i, _):
            degb[pl.ds(i * 16, 16)] = zero16
            return 0
        lax.fori_loop(0, DPW // 16, zd, 0)

        def of(i, _):
            ones_v[pl.ds(i * 16, 16)] = jnp.ones((16,), jnp.float32)
            return 0
        lax.fori_loop(0, CH // 16, of, 0)

        @pl.when(s < WT)
        def _clear():
            for m in range(8):
                sz = 128 if m < 7 else 104
                pltpu.sync_copy(rows.at[0, pl.ds(0, sz)],
                                acc_sh.at[pl.ds(s * RPW + m * 128, sz)])
        if with_deg:
            pltpu.sync_copy(degb, deg_sh.at[pl.ds(s * DPW, DPW)])

        def unpack(k, j):
            def u(i, _):
                p = pbuf[j, pl.ds(i * 16, 16)]
                sidx[k, pl.ds(i * 16, 16)] = p & 16383
                didx[k, pl.ds(i * 16, 16)] = lax.shift_right_logical(p, 14)
                return 0
            lax.fori_loop(0, CH // 16, u, 0)

        for k in range(K):
            unpack(k, k)
            pltpu.async_copy(table_h.at[sidx.at[k]], rows.at[k], sem_g[k])
        plsc.subcore_barrier()

        def group(g, _):
            for k in range(K):
                pltpu.make_async_copy(table_h.at[sidx.at[k]], rows.at[k],
                                      sem_g[k]).wait()
                pltpu.async_copy(rows.at[k], acc_sh.at[didx.at[k]], sem_s[k],
                                 add=True)
                if with_deg:
                    pltpu.async_copy(ones_v, deg_sh.at[didx.at[k]], sem_d[k],
                                     add=True)

                @pl.when(g < NGRP - 1)
                def _refill():
                    pltpu.make_async_copy(rows.at[k], acc_sh.at[didx.at[k]],
                                          sem_s[k]).wait()
                    if with_deg:
                        pltpu.make_async_copy(ones_v, deg_sh.at[didx.at[k]],
                                              sem_d[k]).wait()
                    unpack(k, g * K + k + K)
                    pltpu.async_copy(table_h.at[sidx.at[k]], rows.at[k],
                                     sem_g[k])
            return 0
        lax.fori_loop(0, NGRP, group, 0)
        for k in range(K):
            pltpu.make_async_copy(rows.at[k], acc_sh.at[didx.at[k]],
                                  sem_s[k]).wait()
            if with_deg:
                pltpu.make_async_copy(ones_v, deg_sh.at[didx.at[k]],
                                      sem_d[k]).wait()
        plsc.subcore_barrier()

        @pl.when(s < WT)
        def _writeback():
            pltpu.sync_copy(acc_sh.at[pl.ds(s * RPW, RPW)],
                            acc_h.at[c, pl.ds(s * RPW, RPW)])
        if with_deg:
            pltpu.sync_copy(deg_sh.at[pl.ds(s * DPW, DPW)], degb)
            pltpu.sync_copy(degb, deg_h.at[pl.ds(c * NP_DEG + s * DPW, DPW)])

    res = pl.kernel(
        body,
        out_type=out_type,
        mesh=mesh,
        scratch_types=scratch,
        compiler_params=pltpu.CompilerParams(use_tc_tiling_on_sc=False),
        name=f"sc_agg_w{W}" + ("_deg" if with_deg else ""),
    )(table, pk3)
    return res if with_deg else res[0]


def _mm_t(a, w):
    return lax.dot_general(a, w, (((1,), (1,)), ((), ())),
                           preferred_element_type=jnp.float32)


def _tc_self(h, ws, b):
    H = ws.shape[0]

    def body(x_ref, ws_ref, b_ref, o_ref):
        o_ref[...] = _mm_t(x_ref[...], ws_ref[...]) + b_ref[...]

    return pl.pallas_call(
        body,
        grid=(NB,),
        in_specs=[
            pl.BlockSpec((BN, h.shape[1]), lambda i: (i, 0)),
            pl.BlockSpec(ws.shape, lambda i: (0, 0)),
            pl.BlockSpec((1, H), lambda i: (0, 0)),
        ],
        out_specs=pl.BlockSpec((BN, H), lambda i: (i, 0)),
        out_shape=jax.ShapeDtypeStruct((N, H), jnp.float32),
    )(h, ws, b.reshape(1, H))


def _full(w):
    return pl.BlockSpec(w.shape, lambda i: (0, 0))


def _rows(width):
    return pl.BlockSpec((BN, width), lambda i: (i, 0))


def _mean(d_ref, a_ref):
    d = d_ref[0, 0, :] + d_ref[0, 1, :]
    recip = 1.0 / jnp.maximum(d, 1.0)
    return (a_ref[0] + a_ref[1]) * recip[:, None]


_DEG_SPEC = pl.BlockSpec((1, NC, BN), lambda i: (i, 0, 0))


def _tc_c0(deg3, s0, acc0, wn0, ws1, b1):
    def body(d_ref, s_ref, a_ref, wn_ref, ws_ref, b_ref, h_ref, s1_ref):
        h = jnp.maximum(s_ref[...] + _mm_t(_mean(d_ref, a_ref), wn_ref[...]),
                        0.0)
        h_ref[...] = h
        s1_ref[...] = _mm_t(h, ws_ref[...]) + b_ref[...]

    return pl.pallas_call(
        body,
        grid=(NB,),
        in_specs=[_DEG_SPEC, _rows(H0), pl.BlockSpec((NC, BN, H0),
                  lambda i: (0, i, 0)), _full(wn0), _full(ws1),
                  pl.BlockSpec((1, H0), lambda i: (0, 0))],
        out_specs=[_rows(H0), _rows(H0)],
        out_shape=[jax.ShapeDtypeStruct((N, H0), jnp.float32)] * 2,
    )(deg3, s0, acc0, wn0, ws1, b1.reshape(1, H0))


def _tc_c1(deg3, s1, acc1, wn1, wn2, ws2, b2):
    C = wn2.shape[0]

    def body(d_ref, s_ref, a_ref, wn_ref, wt_ref, ws_ref, b_ref,
             t_ref, s2_ref):
        h = jnp.maximum(s_ref[...] + _mm_t(_mean(d_ref, a_ref), wn_ref[...]),
                        0.0)
        t_ref[...] = _mm_t(h, wt_ref[...])
        s2_ref[...] = _mm_t(h, ws_ref[...]) + b_ref[...]

    return pl.pallas_call(
        body,
        grid=(NB,),
        in_specs=[_DEG_SPEC, _rows(H0), pl.BlockSpec((NC, BN, H0),
                  lambda i: (0, i, 0)), _full(wn1), _full(wn2), _full(ws2),
                  pl.BlockSpec((1, C), lambda i: (0, 0))],
        out_specs=[_rows(C), _rows(C)],
        out_shape=[jax.ShapeDtypeStruct((N, C), jnp.float32)] * 2,
    )(deg3, s1, acc1, wn1, wn2, ws2, b2.reshape(1, C))


def _tc_c2(deg3, s2, acc2):
    C = s2.shape[1]

    def body(d_ref, s_ref, a_ref, o_ref):
        o_ref[...] = s_ref[...] + _mean(d_ref, a_ref)

    return pl.pallas_call(
        body,
        grid=(NB,),
        in_specs=[_DEG_SPEC, _rows(C),
                  pl.BlockSpec((NC, BN, C), lambda i: (0, i, 0))],
        out_specs=_rows(C),
        out_shape=jax.ShapeDtypeStruct((N, C), jnp.float32),
    )(deg3, s2, acc2)


def _tc_pack(edge_index):
    ER = E // CH
    PR = (E_PAD - E) // CH

    def body(e_ref, o_ref):
        o_ref[0:ER, :] = e_ref[0] | (e_ref[1] << 14)
        r = lax.broadcasted_iota(jnp.int32, (PR, CH), 0)
        c = lax.broadcasted_iota(jnp.int32, (PR, CH), 1)
        spread = (r & 1) * 128 + c
        o_ref[ER:ER + PR, :] = spread | ((spread + N) << 14)

    return pl.pallas_call(
        body,
        in_specs=[pl.BlockSpec((2, ER, CH), lambda: (0, 0, 0))],
        out_specs=pl.BlockSpec((ER + PR, CH), lambda: (0, 0)),
        out_shape=jax.ShapeDtypeStruct((ER + PR, CH), jnp.int32),
    )(edge_index.astype(jnp.int32).reshape(2, ER, CH))


def kernel(x, edge_index, W_neigh0, W_self0, b0, W_neigh1, W_self1, b1,
           W_neigh2, W_self2, b2):
    pk3 = _tc_pack(edge_index).reshape(NW, NCHUNK, CH)

    s0 = _tc_self(x, W_self0, b0)
    acc0, degf = _sc_agg(x, pk3, with_deg=True)
    deg3 = degf.reshape(NC, NP_DEG)[:, :N].reshape(NC, NB, BN).transpose(1, 0, 2)

    h0, s1 = _tc_c0(deg3, s0, acc0, W_neigh0, W_self1, b1)
    acc1 = _sc_agg(h0, pk3, with_deg=False)
    t, s2 = _tc_c1(deg3, s1, acc1, W_neigh1, W_neigh2, W_self2, b2)
    acc2 = _sc_agg(t, pk3, with_deg=False)
    return _tc_c2(deg3, s2, acc2)


# speedup vs baseline: 1.2720x; 1.0003x over previous
import jax
import jax.numpy as jnp
from jax import lax
from jax.experimental import pallas as pl
from jax.experimental.pallas import tpu as pltpu
from jax.experimental.pallas import tpu_sc as plsc

N = 10000
E = 320000
NC = 2
NS = 16
NW = NC * NS
CH = 128
NCHUNK = 80
EPW = NCHUNK * CH
E_PAD = NW * EPW
WT = 10
RPW = N // WT
DUMMY = 256
H0 = 128
NP_DEG = 10496
DPW = NP_DEG // NS

BN = 1000
NB = N // BN


def _sc_agg(table, pk3, with_deg):
    W = table.shape[1]
    K = 2 if W == 128 else 4
    NGRP = NCHUNK // K
    mesh = plsc.VectorSubcoreMesh(core_axis_name="c", subcore_axis_name="s",
                                  num_cores=NC, num_subcores=NS)
    out_type = [jax.ShapeDtypeStruct((NC, N, W), jnp.float32)]
    if with_deg:
        out_type.append(jax.ShapeDtypeStruct((NC * NP_DEG,), jnp.float32))
    scratch = [
        pltpu.VMEM((NCHUNK, CH), jnp.int32),
        pltpu.VMEM((K, CH), jnp.int32),
        pltpu.VMEM((K, CH), jnp.int32),
        pltpu.VMEM((K, CH, W), jnp.float32),
        pltpu.VMEM((CH,), jnp.float32),
        pltpu.VMEM((DPW,), jnp.float32),
        pltpu.VMEM_SHARED((N + DUMMY, W), jnp.float32),
        pltpu.VMEM_SHARED((NP_DEG,), jnp.float32),
    ] + [pltpu.SemaphoreType.DMA] * (3 * K)

    def body(table_h, pk_h, acc_h, *rest):
        if with_deg:
            deg_h = rest[0]
            rest = rest[1:]
        pbuf, sidx, didx, rows, ones_v, degb, acc_sh, deg_sh = rest[:8]
        sem_g = rest[8:8 + K]
        sem_s = rest[8 + K:8 + 2 * K]
        sem_d = rest[8 + 2 * K:8 + 3 * K]
        c = lax.axis_index("c")
        s = lax.axis_index("s")
        wid = c * NS + s
        zero16 = jnp.zeros((16,), jnp.float32)
        per_row = W // 16

        pltpu.sync_copy(pk_h.at[wid], pbuf)

        def zr(i, _):
            rows[0, i // per_row, pl.ds((i % per_row) * 16, 16)] = zero16
            return 0
        lax.fori_loop(0, CH * per_row, zr, 0)

        def zd(i, _):
            degb[pl.ds(i * 16, 16)] = zero16
            return 0
        lax.fori_loop(0, DPW // 16, zd, 0)

        def of(i, _):
            ones_v[pl.ds(i * 16, 16)] = jnp.ones((16,), jnp.float32)
            return 0
        lax.fori_loop(0, CH // 16, of, 0)

        @pl.when(s < WT)
        def _clear():
            for m in range(8):
                sz = 128 if m < 7 else 104
                pltpu.sync_copy(rows.at[0, pl.ds(0, sz)],
                                acc_sh.at[pl.ds(s * RPW + m * 128, sz)])
        if with_deg:
            pltpu.sync_copy(degb, deg_sh.at[pl.ds(s * DPW, DPW)])

        def unpack(k, j):
            def u(i, _):
                p = pbuf[j, pl.ds(i * 16, 16)]
                sidx[k, pl.ds(i * 16, 16)] = p & 16383
                didx[k, pl.ds(i * 16, 16)] = lax.shift_right_logical(p, 14)
                return 0
            lax.fori_loop(0, CH // 16, u, 0)

        for k in range(K):
            unpack(k, k)
            pltpu.async_copy(table_h.at[sidx.at[k]], rows.at[k], sem_g[k])
        plsc.subcore_barrier()

        def group(g, _):
            for k in range(K):
                pltpu.make_async_copy(table_h.at[sidx.at[k]], rows.at[k],
                                      sem_g[k]).wait()
                pltpu.async_copy(rows.at[k], acc_sh.at[didx.at[k]], sem_s[k],
                                 add=True)
                if with_deg:
                    pltpu.async_copy(ones_v, deg_sh.at[didx.at[k]], sem_d[k],
                                     add=True)

                @pl.when(g < NGRP - 1)
                def _refill():
                    pltpu.make_async_copy(rows.at[k], acc_sh.at[didx.at[k]],
                                          sem_s[k]).wait()
                    if with_deg:
                        pltpu.make_async_copy(ones_v, deg_sh.at[didx.at[k]],
                                              sem_d[k]).wait()
                    unpack(k, g * K + k + K)
                    pltpu.async_copy(table_h.at[sidx.at[k]], rows.at[k],
                                     sem_g[k])
            return 0
        lax.fori_loop(0, NGRP, group, 0)
        for k in range(K):
            pltpu.make_async_copy(rows.at[k], acc_sh.at[didx.at[k]],
                                  sem_s[k]).wait()
            if with_deg:
                pltpu.make_async_copy(ones_v, deg_sh.at[didx.at[k]],
                                      sem_d[k]).wait()
        plsc.subcore_barrier()

        @pl.when(s < WT)
        def _writeback():
            pltpu.sync_copy(acc_sh.at[pl.ds(s * RPW, RPW)],
                            acc_h.at[c, pl.ds(s * RPW, RPW)])
        if with_deg:
            pltpu.sync_copy(deg_sh.at[pl.ds(s * DPW, DPW)], degb)
            pltpu.sync_copy(degb, deg_h.at[pl.ds(c * NP_DEG + s * DPW, DPW)])

    res = pl.kernel(
        body,
        out_type=out_type,
        mesh=mesh,
        scratch_types=scratch,
        compiler_params=pltpu.CompilerParams(use_tc_tiling_on_sc=False),
        name=f"sc_agg_w{W}" + ("_deg" if with_deg else ""),
    )(table, pk3)
    return res if with_deg else res[0]


def _mm_t(a, w):
    return lax.dot_general(a, w, (((1,), (1,)), ((), ())),
                           preferred_element_type=jnp.float32)


def _tc_self(h, ws, b):
    H = ws.shape[0]

    def body(x_ref, ws_ref, b_ref, o_ref):
        o_ref[...] = _mm_t(x_ref[...], ws_ref[...]) + b_ref[...]

    return pl.pallas_call(
        body,
        grid=(NB,),
        in_specs=[
            pl.BlockSpec((BN, h.shape[1]), lambda i: (i, 0)),
            pl.BlockSpec(ws.shape, lambda i: (0, 0)),
            pl.BlockSpec((1, H), lambda i: (0, 0)),
        ],
        out_specs=pl.BlockSpec((BN, H), lambda i: (i, 0)),
        out_shape=jax.ShapeDtypeStruct((N, H), jnp.float32),
    )(h, ws, b.reshape(1, H))


def _full(w):
    return pl.BlockSpec(w.shape, lambda i: (0, 0))


def _rows(width):
    return pl.BlockSpec((BN, width), lambda i: (i, 0))


def _mean(d_ref, a_ref):
    d = d_ref[0, 0, :] + d_ref[0, 1, :]
    recip = 1.0 / jnp.maximum(d, 1.0)
    return (a_ref[0] + a_ref[1]) * recip[:, None]


_DEG_SPEC = pl.BlockSpec((1, NC, BN), lambda i: (i, 0, 0))


def _tc_c0(deg3, s0, acc0, wn0, ws1, b1):
    def body(d_ref, s_ref, a_ref, wn_ref, ws_ref, b_ref, h_ref, s1_ref):
        h = jnp.maximum(s_ref[...] + _mm_t(_mean(d_ref, a_ref), wn_ref[...]),
                        0.0)
        h_ref[...] = h
        s1_ref[...] = _mm_t(h, ws_ref[...]) + b_ref[...]

    return pl.pallas_call(
        body,
        grid=(NB,),
        in_specs=[_DEG_SPEC, _rows(H0), pl.BlockSpec((NC, BN, H0),
                  lambda i: (0, i, 0)), _full(wn0), _full(ws1),
                  pl.BlockSpec((1, H0), lambda i: (0, 0))],
        out_specs=[_rows(H0), _rows(H0)],
        out_shape=[jax.ShapeDtypeStruct((N, H0), jnp.float32)] * 2,
    )(deg3, s0, acc0, wn0, ws1, b1.reshape(1, H0))


def _tc_c1(deg3, s1, acc1, wn1, wn2, ws2, b2):
    C = wn2.shape[0]

    def body(d_ref, s_ref, a_ref, wn_ref, wt_ref, ws_ref, b_ref,
             t_ref, s2_ref):
        h = jnp.maximum(s_ref[...] + _mm_t(_mean(d_ref, a_ref), wn_ref[...]),
                        0.0)
        t_ref[...] = _mm_t(h, wt_ref[...])
        s2_ref[...] = _mm_t(h, ws_ref[...]) + b_ref[...]

    return pl.pallas_call(
        body,
        grid=(NB,),
        in_specs=[_DEG_SPEC, _rows(H0), pl.BlockSpec((NC, BN, H0),
                  lambda i: (0, i, 0)), _full(wn1), _full(wn2), _full(ws2),
                  pl.BlockSpec((1, C), lambda i: (0, 0))],
        out_specs=[_rows(C), _rows(C)],
        out_shape=[jax.ShapeDtypeStruct((N, C), jnp.float32)] * 2,
    )(deg3, s1, acc1, wn1, wn2, ws2, b2.reshape(1, C))


def _tc_c2(deg3, s2, acc2):
    C = s2.shape[1]

    def body(d_ref, s_ref, a_ref, o_ref):
        o_ref[...] = s_ref[...] + _mean(d_ref, a_ref)

    return pl.pallas_call(
        body,
        grid=(NB,),
        in_specs=[_DEG_SPEC, _rows(C),
                  pl.BlockSpec((NC, BN, C), lambda i: (0, i, 0))],
        out_specs=_rows(C),
        out_shape=jax.ShapeDtypeStruct((N, C), jnp.float32),
    )(deg3, s2, acc2)


def _tc_pack(edge_index):
    ER = E // CH
    PR = (E_PAD - E) // CH

    def body(e_ref, o_ref):
        o_ref[0:ER, :] = e_ref[0] | (e_ref[1] << 14)
        r = lax.broadcasted_iota(jnp.int32, (PR, CH), 0)
        c = lax.broadcasted_iota(jnp.int32, (PR, CH), 1)
        spread = (r & 1) * 128 + c
        o_ref[ER:ER + PR, :] = spread | ((spread + N) << 14)

    return pl.pallas_call(
        body,
        in_specs=[pl.BlockSpec((2, ER, CH), lambda: (0, 0, 0))],
        out_specs=pl.BlockSpec((ER + PR, CH), lambda: (0, 0)),
        out_shape=jax.ShapeDtypeStruct((ER + PR, CH), jnp.int32),
    )(edge_index.astype(jnp.int32).reshape(2, ER, CH))


def kernel(x, edge_index, W_neigh0, W_self0, b0, W_neigh1, W_self1, b1,
           W_neigh2, W_self2, b2):
    pk3 = _tc_pack(edge_index).reshape(NW, NCHUNK, CH)

    s0 = _tc_self(x, W_self0, b0)
    acc0, degf = _sc_agg(x, pk3, with_deg=True)
    deg3 = degf.reshape(NC, NP_DEG)[:, :N].reshape(NC, NB, BN).transpose(1, 0, 2)

    h0, s1 = _tc_c0(deg3, s0, acc0, W_neigh0, W_self1, b1)
    acc1 = _sc_agg(h0, pk3, with_deg=False)
    t, s2 = _tc_c1(deg3, s1, acc1, W_neigh1, W_neigh2, W_self2, b2)
    acc2 = _sc_agg(t, pk3, with_deg=False)
    return _tc_c2(deg3, s2, acc2)
